# baseline (speedup 1.0000x reference)
import functools

import jax
import jax.numpy as jnp
from jax import lax
from jax.experimental import pallas as pl
from jax.experimental.pallas import tpu as pltpu
from jax.experimental.pallas import tpu_sc as plsc

BATCH = 4096
SEQ = 50
NUM_CLASSES = 100
ROWS = BATCH * SEQ
NUM_CORES = 2
NUM_SUBCORES = 16
NW = NUM_CORES * NUM_SUBCORES
B_PER_W = BATCH // NW
CHUNK_B = 16
NUM_CHUNKS = B_PER_W // CHUNK_B
TABLE_PAD = 128

_mesh = plsc.VectorSubcoreMesh(core_axis_name="c", subcore_axis_name="s")


@functools.partial(
    pl.kernel,
    out_type=jax.ShapeDtypeStruct((ROWS,), jnp.int32),
    mesh=_mesh,
    scratch_types=[
        pltpu.VMEM((CHUNK_B * SEQ,), jnp.int32),
        pltpu.VMEM((TABLE_PAD,), jnp.int32),
    ],
    compiler_params=pltpu.CompilerParams(
        needs_layout_passes=False, use_tc_tiling_on_sc=True
    ),
)
def _sc_onehot_to_name(onehot_hbm, table_hbm, out_hbm, outbuf, table_v):
    wid = lax.axis_index("s") * NUM_CORES + lax.axis_index("c")
    b0 = wid * B_PER_W
    pltpu.sync_copy(table_hbm, table_v)
    lanes = lax.iota(jnp.int32, 16)
    out_lanes = lanes * SEQ

    def run(inbuf):
        def chunk_body(ch, carry):
            pltpu.sync_copy(onehot_hbm.at[pl.ds(b0 + ch * CHUNK_B, CHUNK_B)], inbuf)

            def s_body(s, carry2):
                si = jnp.full((16,), 0, jnp.int32) + s
                accs = [jnp.zeros((16,), jnp.float32) for _ in range(4)]
                for c in range(1, NUM_CLASSES):
                    ci = jnp.full((16,), c, jnp.int32)
                    v = plsc.load_gather(inbuf, [lanes, si, ci])
                    accs[c % 4] = accs[c % 4] + v * float(c)
                acc = (accs[0] + accs[1]) + (accs[2] + accs[3])
                names = plsc.load_gather(table_v, [acc.astype(jnp.int32)])
                plsc.store_scatter(outbuf, [out_lanes + s], names)
                return carry2

            lax.fori_loop(0, SEQ, s_body, 0)
            pltpu.sync_copy(
                outbuf, out_hbm.at[pl.ds((b0 + ch * CHUNK_B) * SEQ, CHUNK_B * SEQ)]
            )
            return carry

        lax.fori_loop(0, NUM_CHUNKS, chunk_body, 0)

    pl.run_scoped(run, pltpu.VMEM((CHUNK_B, SEQ, NUM_CLASSES), jnp.float32))


def kernel(onehot, idx_to_name):
    table = jnp.zeros((TABLE_PAD,), jnp.int32).at[:NUM_CLASSES].set(idx_to_name)
    out = _sc_onehot_to_name(onehot, table)
    return out.reshape(BATCH, SEQ)


# speedup vs baseline: 2.4285x; 2.4285x over previous
import functools

import jax
import jax.numpy as jnp
from jax import lax
from jax.experimental import pallas as pl
from jax.experimental.pallas import tpu as pltpu
from jax.experimental.pallas import tpu_sc as plsc

BATCH = 4096
SEQ = 50
NUM_CLASSES = 100
NUM_CORES = 2
NUM_SUBCORES = 16
NW = NUM_CORES * NUM_SUBCORES
TABLE_PAD = 128

B_SC = 512
B_TC = BATCH - B_SC
B_PER_W = B_SC // NW
CHUNK_B = 16
NUM_CHUNKS = B_PER_W // CHUNK_B

TC_BB = 64
TC_ROWS = TC_BB * SEQ
TC_NB = B_TC // TC_BB

_mesh = plsc.VectorSubcoreMesh(core_axis_name="c", subcore_axis_name="s")


@functools.partial(
    pl.kernel,
    out_type=jax.ShapeDtypeStruct((B_SC * SEQ,), jnp.int32),
    mesh=_mesh,
    scratch_types=[
        pltpu.VMEM((CHUNK_B * SEQ,), jnp.int32),
        pltpu.VMEM((TABLE_PAD,), jnp.int32),
    ],
    compiler_params=pltpu.CompilerParams(
        needs_layout_passes=False, use_tc_tiling_on_sc=True
    ),
)
def _sc_part(onehot_hbm, table_hbm, out_hbm, outbuf, table_v):
    wid = lax.axis_index("s") * NUM_CORES + lax.axis_index("c")
    b0 = wid * B_PER_W
    pltpu.sync_copy(table_hbm, table_v)
    lanes = lax.iota(jnp.int32, 16)
    out_lanes = lanes * SEQ

    def run(inbuf):
        def chunk_body(ch, carry):
            pltpu.sync_copy(onehot_hbm.at[pl.ds(b0 + ch * CHUNK_B, CHUNK_B)], inbuf)

            def s_body(s, carry2):
                si = jnp.full((16,), 0, jnp.int32) + s
                accs = [jnp.zeros((16,), jnp.float32) for _ in range(4)]
                for c in range(1, NUM_CLASSES):
                    ci = jnp.full((16,), c, jnp.int32)
                    v = plsc.load_gather(inbuf, [lanes, si, ci])
                    accs[c % 4] = accs[c % 4] + v * float(c)
                acc = (accs[0] + accs[1]) + (accs[2] + accs[3])
                names = plsc.load_gather(table_v, [acc.astype(jnp.int32)])
                plsc.store_scatter(outbuf, [out_lanes + s], names)
                return carry2

            lax.fori_loop(0, SEQ, s_body, 0)
            pltpu.sync_copy(
                outbuf, out_hbm.at[pl.ds((b0 + ch * CHUNK_B) * SEQ, CHUNK_B * SEQ)]
            )
            return carry

        lax.fori_loop(0, NUM_CHUNKS, chunk_body, 0)

    pl.run_scoped(run, pltpu.VMEM((CHUNK_B, SEQ, NUM_CLASSES), jnp.float32))


def _tc_body(x_ref, w_ref, o_ref):
    x2 = x_ref[...].reshape(TC_ROWS, NUM_CLASSES)
    w = w_ref[...]
    z = lax.dot_general(
        w, x2, (((1,), (1,)), ((), ())), preferred_element_type=jnp.float32
    )
    o_ref[...] = z.astype(jnp.int32).reshape(1, 1, TC_ROWS)


_tc_part = pl.pallas_call(
    _tc_body,
    grid=(TC_NB,),
    in_specs=[
        pl.BlockSpec(
            (TC_BB, SEQ, NUM_CLASSES), lambda i: (B_SC // TC_BB + i, 0, 0)
        ),
        pl.BlockSpec((1, NUM_CLASSES), lambda i: (0, 0)),
    ],
    out_specs=pl.BlockSpec((1, 1, TC_ROWS), lambda i: (i, 0, 0)),
    out_shape=jax.ShapeDtypeStruct((TC_NB, 1, TC_ROWS), jnp.int32),
)


def kernel(onehot, idx_to_name):
    table_i = jnp.zeros((TABLE_PAD,), jnp.int32).at[:NUM_CLASSES].set(idx_to_name)
    w = idx_to_name.astype(jnp.float32).reshape(1, NUM_CLASSES)
    sc_out = _sc_part(onehot, table_i)
    tc_out = _tc_part(onehot, w)
    return jnp.concatenate(
        [sc_out.reshape(B_SC, SEQ), tc_out.reshape(B_TC, SEQ)], axis=0
    )


# speedup vs baseline: 2.4658x; 1.0154x over previous
import functools

import jax
import jax.numpy as jnp
from jax import lax
from jax.experimental import pallas as pl
from jax.experimental.pallas import tpu as pltpu
from jax.experimental.pallas import tpu_sc as plsc

BATCH = 4096
SEQ = 50
NUM_CLASSES = 100
NUM_CORES = 2
NUM_SUBCORES = 16
NW = NUM_CORES * NUM_SUBCORES
TABLE_PAD = 128

B_SC = 512
B_TC = BATCH - B_SC
B_PER_W = B_SC // NW
CHUNK_B = 16
NUM_CHUNKS = B_PER_W // CHUNK_B

TC_BB = 128
TC_ROWS = TC_BB * SEQ
TC_NB = B_TC // TC_BB

_mesh = plsc.VectorSubcoreMesh(core_axis_name="c", subcore_axis_name="s")


@functools.partial(
    pl.kernel,
    out_type=jax.ShapeDtypeStruct((B_SC * SEQ,), jnp.int32),
    mesh=_mesh,
    scratch_types=[
        pltpu.VMEM((CHUNK_B * SEQ,), jnp.int32),
        pltpu.VMEM((TABLE_PAD,), jnp.int32),
    ],
    compiler_params=pltpu.CompilerParams(
        needs_layout_passes=False, use_tc_tiling_on_sc=True
    ),
)
def _sc_part(onehot_hbm, table_hbm, out_hbm, outbuf, table_v):
    wid = lax.axis_index("s") * NUM_CORES + lax.axis_index("c")
    b0 = wid * B_PER_W
    pltpu.sync_copy(table_hbm, table_v)
    lanes = lax.iota(jnp.int32, 16)
    out_lanes = lanes * SEQ

    def run(inbuf):
        def chunk_body(ch, carry):
            pltpu.sync_copy(onehot_hbm.at[pl.ds(b0 + ch * CHUNK_B, CHUNK_B)], inbuf)

            def s_body(s, carry2):
                si = jnp.full((16,), 0, jnp.int32) + s
                accs = [jnp.zeros((16,), jnp.float32) for _ in range(4)]
                for c in range(1, NUM_CLASSES):
                    ci = jnp.full((16,), c, jnp.int32)
                    v = plsc.load_gather(inbuf, [lanes, si, ci])
                    accs[c % 4] = accs[c % 4] + v * float(c)
                acc = (accs[0] + accs[1]) + (accs[2] + accs[3])
                names = plsc.load_gather(table_v, [acc.astype(jnp.int32)])
                plsc.store_scatter(outbuf, [out_lanes + s], names)
                return carry2

            lax.fori_loop(0, SEQ, s_body, 0)
            pltpu.sync_copy(
                outbuf, out_hbm.at[pl.ds((b0 + ch * CHUNK_B) * SEQ, CHUNK_B * SEQ)]
            )
            return carry

        lax.fori_loop(0, NUM_CHUNKS, chunk_body, 0)

    pl.run_scoped(run, pltpu.VMEM((CHUNK_B, SEQ, NUM_CLASSES), jnp.float32))


def _tc_body(x_ref, w_ref, o_ref):
    x2 = x_ref[...].reshape(TC_ROWS, NUM_CLASSES)
    w = w_ref[...]
    z = lax.dot_general(
        w, x2, (((1,), (1,)), ((), ())), preferred_element_type=jnp.float32
    )
    o_ref[...] = z.astype(jnp.int32).reshape(1, 1, TC_ROWS)


_tc_part = pl.pallas_call(
    _tc_body,
    grid=(TC_NB,),
    in_specs=[
        pl.BlockSpec(
            (TC_BB, SEQ, NUM_CLASSES), lambda i: (B_SC // TC_BB + i, 0, 0)
        ),
        pl.BlockSpec((1, NUM_CLASSES), lambda i: (0, 0)),
    ],
    out_specs=pl.BlockSpec((1, 1, TC_ROWS), lambda i: (i, 0, 0)),
    out_shape=jax.ShapeDtypeStruct((TC_NB, 1, TC_ROWS), jnp.int32),
)


def kernel(onehot, idx_to_name):
    table_i = jnp.zeros((TABLE_PAD,), jnp.int32).at[:NUM_CLASSES].set(idx_to_name)
    w = idx_to_name.astype(jnp.float32).reshape(1, NUM_CLASSES)
    sc_in = lax.slice(onehot, (0, 0, 0), (B_SC, SEQ, NUM_CLASSES))
    sc_out = _sc_part(sc_in, table_i)
    tc_out = _tc_part(onehot, w)
    return jnp.concatenate(
        [sc_out.reshape(B_SC, SEQ), tc_out.reshape(B_TC, SEQ)], axis=0
    )


# speedup vs baseline: 4.9710x; 2.0159x over previous
import functools

import jax
import jax.numpy as jnp
from jax import lax
from jax.experimental import pallas as pl
from jax.experimental.pallas import tpu as pltpu
from jax.experimental.pallas import tpu_sc as plsc

BATCH = 4096
SEQ = 50
NUM_CLASSES = 100
NUM_CORES = 2
NUM_SUBCORES = 16
NW = NUM_CORES * NUM_SUBCORES
TABLE_PAD = 128

B_SC = 512
B_TC = BATCH - B_SC
B_PER_W = B_SC // NW
CHUNK_B = 16
NUM_CHUNKS = B_PER_W // CHUNK_B

TC_BB = 128
TC_NB = B_TC // TC_BB

_mesh = plsc.VectorSubcoreMesh(core_axis_name="c", subcore_axis_name="s")


@functools.partial(
    pl.kernel,
    out_type=jax.ShapeDtypeStruct((B_SC * SEQ,), jnp.int32),
    mesh=_mesh,
    scratch_types=[
        pltpu.VMEM((CHUNK_B * SEQ,), jnp.int32),
        pltpu.VMEM((TABLE_PAD,), jnp.int32),
    ],
    compiler_params=pltpu.CompilerParams(
        needs_layout_passes=False, use_tc_tiling_on_sc=True
    ),
)
def _sc_part(onehot_hbm, table_hbm, out_hbm, outbuf, table_v):
    wid = lax.axis_index("s") * NUM_CORES + lax.axis_index("c")
    b0 = wid * B_PER_W
    pltpu.sync_copy(table_hbm, table_v)
    lanes = lax.iota(jnp.int32, 16)
    out_lanes = lanes * SEQ

    def run(inbuf):
        def chunk_body(ch, carry):
            pltpu.sync_copy(onehot_hbm.at[pl.ds(b0 + ch * CHUNK_B, CHUNK_B)], inbuf)

            def s_body(s, carry2):
                si = jnp.full((16,), 0, jnp.int32) + s
                accs = [jnp.zeros((16,), jnp.float32) for _ in range(4)]
                for c in range(1, NUM_CLASSES):
                    ci = jnp.full((16,), c, jnp.int32)
                    v = plsc.load_gather(inbuf, [lanes, si, ci])
                    accs[c % 4] = accs[c % 4] + v * float(c)
                acc = (accs[0] + accs[1]) + (accs[2] + accs[3])
                names = plsc.load_gather(table_v, [acc.astype(jnp.int32)])
                plsc.store_scatter(outbuf, [out_lanes + s], names)
                return carry2

            lax.fori_loop(0, SEQ, s_body, 0)
            pltpu.sync_copy(
                outbuf, out_hbm.at[pl.ds((b0 + ch * CHUNK_B) * SEQ, CHUNK_B * SEQ)]
            )
            return carry

        lax.fori_loop(0, NUM_CHUNKS, chunk_body, 0)

    pl.run_scoped(run, pltpu.VMEM((CHUNK_B, SEQ, NUM_CLASSES), jnp.float32))


def _tc_body(x_ref, w_ref, o_ref):
    x = x_ref[...]
    w = w_ref[...]
    o_ref[...] = jnp.sum(x * w, axis=1).astype(jnp.int32)


_tc_part = pl.pallas_call(
    _tc_body,
    grid=(TC_NB,),
    in_specs=[
        pl.BlockSpec(
            (SEQ, NUM_CLASSES, TC_BB), lambda i: (0, 0, B_SC // TC_BB + i)
        ),
        pl.BlockSpec((1, NUM_CLASSES, TC_BB), lambda i: (0, 0, 0)),
    ],
    out_specs=pl.BlockSpec((SEQ, TC_BB), lambda i: (0, i)),
    out_shape=jax.ShapeDtypeStruct((SEQ, B_TC), jnp.int32),
)


def kernel(onehot, idx_to_name):
    table_i = jnp.zeros((TABLE_PAD,), jnp.int32).at[:NUM_CLASSES].set(idx_to_name)
    wf = idx_to_name.astype(jnp.float32)
    wtile = wf.reshape(1, NUM_CLASSES, 1) * jnp.ones((1, 1, TC_BB), jnp.float32)
    onehot_t = jnp.transpose(onehot, (1, 2, 0))
    sc_in = lax.slice(onehot, (0, 0, 0), (B_SC, SEQ, NUM_CLASSES))
    sc_out = _sc_part(sc_in, table_i)
    tc_out_t = _tc_part(onehot_t, wtile)
    sc_out_t = sc_out.reshape(B_SC, SEQ).T
    out_t = jnp.concatenate([sc_out_t, tc_out_t], axis=1)
    return jnp.transpose(out_t, (1, 0))


# speedup vs baseline: 7.1536x; 1.4391x over previous
import functools

import jax
import jax.numpy as jnp
from jax import lax
from jax.experimental import pallas as pl
from jax.experimental.pallas import tpu as pltpu
from jax.experimental.pallas import tpu_sc as plsc

BATCH = 4096
SEQ = 50
NUM_CLASSES = 100
NUM_CORES = 2
NUM_SUBCORES = 16
NW = NUM_CORES * NUM_SUBCORES
TABLE_PAD = 128

B_SC = 256
B_TC = BATCH - B_SC
B_PER_W = B_SC // NW
CHUNK_B = 8
NUM_CHUNKS = B_PER_W // CHUNK_B

TC_BB = 128
TC_NB = B_TC // TC_BB

_mesh = plsc.VectorSubcoreMesh(core_axis_name="c", subcore_axis_name="s")


@functools.partial(
    pl.kernel,
    out_type=jax.ShapeDtypeStruct((B_SC * SEQ,), jnp.int32),
    mesh=_mesh,
    scratch_types=[
        pltpu.VMEM((CHUNK_B * SEQ,), jnp.int32),
        pltpu.VMEM((TABLE_PAD,), jnp.int32),
    ],
    compiler_params=pltpu.CompilerParams(
        needs_layout_passes=False, use_tc_tiling_on_sc=True
    ),
)
def _sc_part(onehot_hbm, table_hbm, out_hbm, outbuf, table_v):
    wid = lax.axis_index("s") * NUM_CORES + lax.axis_index("c")
    b0 = wid * B_PER_W
    pltpu.sync_copy(table_hbm, table_v)
    lanes = lax.iota(jnp.int32, 16)
    bi = lanes & 7
    s_off = lanes >> 3
    out_base = bi * SEQ + s_off

    def run(inbuf):
        def chunk_body(ch, carry):
            pltpu.sync_copy(onehot_hbm.at[pl.ds(b0 + ch * CHUNK_B, CHUNK_B)], inbuf)

            def s_body(g, carry2):
                si = s_off + 2 * g
                accs = [jnp.zeros((16,), jnp.float32) for _ in range(4)]
                for c in range(1, NUM_CLASSES):
                    ci = jnp.full((16,), c, jnp.int32)
                    v = plsc.load_gather(inbuf, [bi, si, ci])
                    accs[c % 4] = accs[c % 4] + v * float(c)
                acc = (accs[0] + accs[1]) + (accs[2] + accs[3])
                names = plsc.load_gather(table_v, [acc.astype(jnp.int32)])
                plsc.store_scatter(outbuf, [out_base + 2 * g], names)
                return carry2

            lax.fori_loop(0, SEQ // 2, s_body, 0)
            pltpu.sync_copy(
                outbuf, out_hbm.at[pl.ds((b0 + ch * CHUNK_B) * SEQ, CHUNK_B * SEQ)]
            )
            return carry

        lax.fori_loop(0, NUM_CHUNKS, chunk_body, 0)

    pl.run_scoped(run, pltpu.VMEM((CHUNK_B, SEQ, NUM_CLASSES), jnp.float32))


def _tc_body(x_ref, w_ref, o_ref):
    x = x_ref[...]
    w = w_ref[...]
    o_ref[...] = jnp.sum(x * w, axis=1).astype(jnp.int32)


_tc_part = pl.pallas_call(
    _tc_body,
    grid=(TC_NB,),
    in_specs=[
        pl.BlockSpec(
            (SEQ, NUM_CLASSES, TC_BB), lambda i: (0, 0, B_SC // TC_BB + i)
        ),
        pl.BlockSpec((1, NUM_CLASSES, TC_BB), lambda i: (0, 0, 0)),
    ],
    out_specs=pl.BlockSpec((SEQ, TC_BB), lambda i: (0, i)),
    out_shape=jax.ShapeDtypeStruct((SEQ, B_TC), jnp.int32),
)


def kernel(onehot, idx_to_name):
    table_i = jnp.zeros((TABLE_PAD,), jnp.int32).at[:NUM_CLASSES].set(idx_to_name)
    wf = idx_to_name.astype(jnp.float32)
    wtile = wf.reshape(1, NUM_CLASSES, 1) * jnp.ones((1, 1, TC_BB), jnp.float32)
    onehot_t = jnp.transpose(onehot, (1, 2, 0))
    sc_in = lax.slice(onehot, (0, 0, 0), (B_SC, SEQ, NUM_CLASSES))
    sc_out = _sc_part(sc_in, table_i)
    tc_out_t = _tc_part(onehot_t, wtile)
    sc_out_t = sc_out.reshape(B_SC, SEQ).T
    out_t = jnp.concatenate([sc_out_t, tc_out_t], axis=1)
    return jnp.transpose(out_t, (1, 0))


# speedup vs baseline: 7.4410x; 1.0402x over previous
import functools

import jax
import jax.numpy as jnp
from jax import lax
from jax.experimental import pallas as pl
from jax.experimental.pallas import tpu as pltpu
from jax.experimental.pallas import tpu_sc as plsc

BATCH = 4096
SEQ = 50
NUM_CLASSES = 100
NUM_CORES = 2
NUM_SUBCORES = 16
NW = NUM_CORES * NUM_SUBCORES
TABLE_PAD = 128

S_TC = 40
S_SC = SEQ - S_TC
SC_CH_S = 5
SC_BW = BATCH // NW

TC_SB = 8
TC_BB = 512
TC_GS = S_TC // TC_SB
TC_GB = BATCH // TC_BB

_mesh = plsc.VectorSubcoreMesh(core_axis_name="c", subcore_axis_name="s")


@functools.partial(
    pl.kernel,
    out_type=jax.ShapeDtypeStruct((S_SC, BATCH), jnp.int32),
    mesh=_mesh,
    scratch_types=[
        pltpu.VMEM((S_SC, SC_BW), jnp.int32),
        pltpu.VMEM((TABLE_PAD,), jnp.int32),
    ],
    compiler_params=pltpu.CompilerParams(
        needs_layout_passes=False, use_tc_tiling_on_sc=True
    ),
)
def _sc_part(onehot_t_hbm, table_hbm, out_hbm, outbuf, table_v):
    wid = lax.axis_index("s") * NUM_CORES + lax.axis_index("c")
    b0 = wid * SC_BW
    pltpu.sync_copy(table_hbm, table_v)

    def run(inbuf):
        for ch in range(S_SC // SC_CH_S):
            pltpu.sync_copy(
                onehot_t_hbm.at[
                    pl.ds(S_TC + ch * SC_CH_S, SC_CH_S), :, pl.ds(b0, SC_BW)
                ],
                inbuf,
            )

            def bl_body(bl, carry, ch=ch):
                bsl = pl.ds(bl * 16, 16)
                for s_l in range(SC_CH_S):
                    accs = [jnp.zeros((16,), jnp.float32) for _ in range(4)]
                    for c in range(1, NUM_CLASSES):
                        v = inbuf[s_l, c, bsl]
                        accs[c % 4] = accs[c % 4] + v * float(c)
                    acc = (accs[0] + accs[1]) + (accs[2] + accs[3])
                    names = plsc.load_gather(table_v, [acc.astype(jnp.int32)])
                    outbuf[ch * SC_CH_S + s_l, bsl] = names
                return carry

            lax.fori_loop(0, SC_BW // 16, bl_body, 0)

    pl.run_scoped(run, pltpu.VMEM((SC_CH_S, NUM_CLASSES, SC_BW), jnp.float32))
    pltpu.sync_copy(outbuf, out_hbm.at[:, pl.ds(b0, SC_BW)])


def _tc_body(x_ref, w_ref, o_ref):
    x = x_ref[...]
    w = w_ref[...]
    o_ref[...] = jnp.sum(x * w, axis=1).astype(jnp.int32)


_tc_part = pl.pallas_call(
    _tc_body,
    grid=(TC_GS, TC_GB),
    in_specs=[
        pl.BlockSpec((TC_SB, NUM_CLASSES, TC_BB), lambda i, j: (i, 0, j)),
        pl.BlockSpec((1, NUM_CLASSES, TC_BB), lambda i, j: (0, 0, 0)),
    ],
    out_specs=pl.BlockSpec((TC_SB, TC_BB), lambda i, j: (i, j)),
    out_shape=jax.ShapeDtypeStruct((S_TC, BATCH), jnp.int32),
)


def kernel(onehot, idx_to_name):
    table_i = jnp.zeros((TABLE_PAD,), jnp.int32).at[:NUM_CLASSES].set(idx_to_name)
    wf = idx_to_name.astype(jnp.float32)
    wtile = wf.reshape(1, NUM_CLASSES, 1) * jnp.ones((1, 1, TC_BB), jnp.float32)
    onehot_t = jnp.transpose(onehot, (1, 2, 0))
    sc_out = _sc_part(onehot_t, table_i)
    tc_out = _tc_part(onehot_t, wtile)
    out_t = jnp.concatenate([tc_out, sc_out], axis=0)
    return jnp.transpose(out_t, (1, 0))


# speedup vs baseline: 8.2017x; 1.1022x over previous
import functools

import jax
import jax.numpy as jnp
from jax import lax
from jax.experimental import pallas as pl
from jax.experimental.pallas import tpu as pltpu
from jax.experimental.pallas import tpu_sc as plsc

BATCH = 4096
SEQ = 50
NUM_CLASSES = 100
NUM_CORES = 2
NUM_SUBCORES = 16
NW = NUM_CORES * NUM_SUBCORES
TABLE_PAD = 128

S_TC = 32
S_SC = SEQ - S_TC
SC_CH_S = 6
SC_BW = BATCH // NW

TC_SB = 8
TC_BB = 512
TC_GS = S_TC // TC_SB
TC_GB = BATCH // TC_BB

_mesh = plsc.VectorSubcoreMesh(core_axis_name="c", subcore_axis_name="s")


@functools.partial(
    pl.kernel,
    out_type=jax.ShapeDtypeStruct((S_SC, BATCH), jnp.int32),
    mesh=_mesh,
    scratch_types=[
        pltpu.VMEM((S_SC, SC_BW), jnp.int32),
        pltpu.VMEM((TABLE_PAD,), jnp.int32),
    ],
    compiler_params=pltpu.CompilerParams(
        needs_layout_passes=False, use_tc_tiling_on_sc=True
    ),
)
def _sc_part(onehot_t_hbm, table_hbm, out_hbm, outbuf, table_v):
    wid = lax.axis_index("s") * NUM_CORES + lax.axis_index("c")
    b0 = wid * SC_BW
    pltpu.sync_copy(table_hbm, table_v)

    def run(inbuf):
        for ch in range(S_SC // SC_CH_S):
            pltpu.sync_copy(
                onehot_t_hbm.at[
                    pl.ds(S_TC + ch * SC_CH_S, SC_CH_S), :, pl.ds(b0, SC_BW)
                ],
                inbuf,
            )

            def bl_body(bl, carry, ch=ch):
                bsl = pl.ds(bl * 16, 16)
                for s_l in range(SC_CH_S):
                    accs = [jnp.zeros((16,), jnp.float32) for _ in range(4)]
                    for c in range(1, NUM_CLASSES):
                        v = inbuf[s_l, c, bsl]
                        accs[c % 4] = accs[c % 4] + v * float(c)
                    acc = (accs[0] + accs[1]) + (accs[2] + accs[3])
                    names = plsc.load_gather(table_v, [acc.astype(jnp.int32)])
                    outbuf[ch * SC_CH_S + s_l, bsl] = names
                return carry

            lax.fori_loop(0, SC_BW // 16, bl_body, 0)

    pl.run_scoped(run, pltpu.VMEM((SC_CH_S, NUM_CLASSES, SC_BW), jnp.float32))
    pltpu.sync_copy(outbuf, out_hbm.at[:, pl.ds(b0, SC_BW)])


def _tc_body(x_ref, w_ref, o_ref):
    x = x_ref[...]
    w = w_ref[...]
    o_ref[...] = jnp.sum(x * w, axis=1).astype(jnp.int32)


_tc_part = pl.pallas_call(
    _tc_body,
    grid=(TC_GS, TC_GB),
    in_specs=[
        pl.BlockSpec((TC_SB, NUM_CLASSES, TC_BB), lambda i, j: (i, 0, j)),
        pl.BlockSpec((1, NUM_CLASSES, TC_BB), lambda i, j: (0, 0, 0)),
    ],
    out_specs=pl.BlockSpec((TC_SB, TC_BB), lambda i, j: (i, j)),
    out_shape=jax.ShapeDtypeStruct((S_TC, BATCH), jnp.int32),
)


def kernel(onehot, idx_to_name):
    table_i = jnp.zeros((TABLE_PAD,), jnp.int32).at[:NUM_CLASSES].set(idx_to_name)
    wf = idx_to_name.astype(jnp.float32)
    wtile = wf.reshape(1, NUM_CLASSES, 1) * jnp.ones((1, 1, TC_BB), jnp.float32)
    onehot_t = jnp.transpose(onehot, (1, 2, 0))
    sc_out = _sc_part(onehot_t, table_i)
    tc_out = _tc_part(onehot_t, wtile)
    out_t = jnp.concatenate([tc_out, sc_out], axis=0)
    return jnp.transpose(out_t, (1, 0))


# speedup vs baseline: 8.3491x; 1.0180x over previous
import functools

import jax
import jax.numpy as jnp
from jax import lax
from jax.experimental import pallas as pl
from jax.experimental.pallas import tpu as pltpu
from jax.experimental.pallas import tpu_sc as plsc

BATCH = 4096
SEQ = 50
NUM_CLASSES = 100
NUM_CORES = 2
NUM_SUBCORES = 16
NW = NUM_CORES * NUM_SUBCORES
TABLE_PAD = 128

S_TC = 32
S_SC = SEQ - S_TC
SC_CH_S = 6
SC_BW = BATCH // NW

TC_SB = 8
TC_BB = 512
TC_GS = S_TC // TC_SB
TC_GB = BATCH // TC_BB

_mesh = plsc.VectorSubcoreMesh(core_axis_name="c", subcore_axis_name="s")


@functools.partial(
    pl.kernel,
    out_type=jax.ShapeDtypeStruct((S_SC, BATCH), jnp.int32),
    mesh=_mesh,
    scratch_types=[
        pltpu.VMEM((S_SC, SC_BW), jnp.int32),
        pltpu.VMEM((TABLE_PAD,), jnp.int32),
    ],
    compiler_params=pltpu.CompilerParams(
        needs_layout_passes=False, use_tc_tiling_on_sc=True
    ),
)
def _sc_part(onehot_t_hbm, table_hbm, out_hbm, outbuf, table_v):
    wid = lax.axis_index("s") * NUM_CORES + lax.axis_index("c")
    b0 = wid * SC_BW
    pltpu.sync_copy(table_hbm, table_v)

    def run(inbuf):
        for ch in range(S_SC // SC_CH_S):
            pltpu.sync_copy(
                onehot_t_hbm.at[
                    pl.ds(S_TC + ch * SC_CH_S, SC_CH_S), :, pl.ds(b0, SC_BW)
                ],
                inbuf,
            )

            def bl_body(bl, carry, ch=ch):
                bsl = pl.ds(bl * 16, 16)
                for s_l in range(SC_CH_S):
                    accs = [jnp.zeros((16,), jnp.float32) for _ in range(4)]
                    for c in range(1, NUM_CLASSES):
                        v = inbuf[s_l, c, bsl]
                        accs[c % 4] = accs[c % 4] + v * float(c)
                    acc = (accs[0] + accs[1]) + (accs[2] + accs[3])
                    names = plsc.load_gather(table_v, [acc.astype(jnp.int32)])
                    outbuf[ch * SC_CH_S + s_l, bsl] = names
                return carry

            lax.fori_loop(0, SC_BW // 16, bl_body, 0)

    pl.run_scoped(run, pltpu.VMEM((SC_CH_S, NUM_CLASSES, SC_BW), jnp.float32))
    pltpu.sync_copy(outbuf, out_hbm.at[:, pl.ds(b0, SC_BW)])


def _tc_body(x_ref, w_ref, o_ref):
    w = w_ref[...]
    for si in range(TC_SB):
        z = lax.dot_general(
            w, x_ref[si], (((1,), (0,)), ((), ())),
            preferred_element_type=jnp.float32,
        )
        o_ref[si, :] = z.reshape(TC_BB).astype(jnp.int32)


_tc_part = pl.pallas_call(
    _tc_body,
    grid=(TC_GS, TC_GB),
    in_specs=[
        pl.BlockSpec((TC_SB, NUM_CLASSES, TC_BB), lambda i, j: (i, 0, j)),
        pl.BlockSpec((1, NUM_CLASSES), lambda i, j: (0, 0)),
    ],
    out_specs=pl.BlockSpec((TC_SB, TC_BB), lambda i, j: (i, j)),
    out_shape=jax.ShapeDtypeStruct((S_TC, BATCH), jnp.int32),
)


def kernel(onehot, idx_to_name):
    table_i = jnp.zeros((TABLE_PAD,), jnp.int32).at[:NUM_CLASSES].set(idx_to_name)
    wtile = idx_to_name.astype(jnp.float32).reshape(1, NUM_CLASSES)
    onehot_t = jnp.transpose(onehot, (1, 2, 0))
    sc_out = _sc_part(onehot_t, table_i)
    tc_out = _tc_part(onehot_t, wtile)
    out_t = jnp.concatenate([tc_out, sc_out], axis=0)
    return jnp.transpose(out_t, (1, 0))
